# baseline (speedup 1.0000x reference)
import functools

import jax
import jax.numpy as jnp
from jax import lax
from jax.experimental import pallas as pl
from jax.experimental.pallas import tpu as pltpu
from jax.experimental.pallas import tpu_sc as plsc

CHUNK = 400
N_CORES = 2
N_SUBCORES = 16
N_WORKERS = N_CORES * N_SUBCORES


def _mesh():
    return plsc.VectorSubcoreMesh(core_axis_name="c", subcore_axis_name="s",
                                  num_cores=N_CORES, num_subcores=N_SUBCORES)


_SC_PARAMS = pltpu.CompilerParams(use_tc_tiling_on_sc=False)


def _make_deg(n_pad, e, chunk):
    CHUNK = chunk
    per_w = e // CHUNK // N_WORKERS

    @functools.partial(
        pl.kernel,
        out_type=jax.ShapeDtypeStruct((N_CORES, n_pad, 8), jnp.float32),
        mesh=_mesh(),
        scratch_types=[
            pltpu.VMEM((CHUNK,), jnp.int32),
            pltpu.VMEM((CHUNK,), jnp.int32),
            pltpu.VMEM((CHUNK, 8), jnp.float32),
            pltpu.VMEM_SHARED((n_pad, 8), jnp.float32),
            pltpu.SemaphoreType.DMA,
            pltpu.SemaphoreType.DMA,
        ],
        compiler_params=_SC_PARAMS,
    )
    def deg_kernel(dst_hbm, ones_hbm, zcol_hbm, out_hbm,
                   didx0, didx1, ones_v, acc, isem0, isem1):
        cid = lax.axis_index("c")
        sid = lax.axis_index("s")
        wid = sid * N_CORES + cid
        rpt = n_pad // N_SUBCORES
        r0 = sid * rpt
        didx = (didx0, didx1)
        isem = (isem0, isem1)

        pltpu.sync_copy(zcol_hbm.at[pl.ds(r0, rpt)], acc.at[pl.ds(r0, rpt)])
        pltpu.sync_copy(ones_hbm, ones_v)

        def b(j):
            return pl.multiple_of((wid * per_w + j) * CHUNK, 8)

        descs = {0: pltpu.async_copy(dst_hbm.at[pl.ds(b(0), CHUNK)],
                                     didx[0], isem[0])}
        plsc.subcore_barrier()
        for j in range(per_w):
            cur = j & 1
            if j + 1 < per_w:
                descs[j + 1] = pltpu.async_copy(
                    dst_hbm.at[pl.ds(b(j + 1), CHUNK)], didx[1 - cur],
                    isem[1 - cur])
            descs.pop(j).wait()
            pltpu.sync_copy(ones_v, acc.at[didx[cur]], add=True)

        plsc.subcore_barrier()
        pltpu.sync_copy(acc.at[pl.ds(r0, rpt)],
                        out_hbm.at[cid, pl.ds(r0, rpt)])

    return deg_kernel


def _make_agg(n_pad, w, e, chunk):
    CHUNK = chunk
    n_chunks = e // CHUNK
    per_lo = n_chunks // N_WORKERS
    n_extra = n_chunks - per_lo * N_WORKERS
    per_hi = per_lo + (1 if n_extra else 0)

    @functools.partial(
        pl.kernel,
        out_type=jax.ShapeDtypeStruct((N_CORES, n_pad, w), jnp.float32),
        mesh=_mesh(),
        scratch_types=[
            pltpu.VMEM((CHUNK,), jnp.int32),
            pltpu.VMEM((CHUNK,), jnp.int32),
            pltpu.VMEM((CHUNK,), jnp.int32),
            pltpu.VMEM((CHUNK,), jnp.int32),
            pltpu.VMEM((CHUNK, w), jnp.float32),
            pltpu.VMEM((CHUNK, w), jnp.float32),
            pltpu.VMEM_SHARED((n_pad, w), jnp.float32),
            pltpu.SemaphoreType.DMA,
            pltpu.SemaphoreType.DMA,
            pltpu.SemaphoreType.DMA,
            pltpu.SemaphoreType.DMA,
        ],
        compiler_params=_SC_PARAMS,
    )
    def agg_kernel(src_hbm, dst_hbm, vals_hbm, zeros_hbm, out_hbm,
                   sidx0, sidx1, didx0, didx1, rows0, rows1, acc,
                   gsem0, gsem1, isem0, isem1):
        cid = lax.axis_index("c")
        sid = lax.axis_index("s")
        wid = sid * N_CORES + cid
        rpt = n_pad // N_SUBCORES
        r0 = sid * rpt
        sidx = (sidx0, sidx1)
        didx = (didx0, didx1)
        rows = (rows0, rows1)
        gsem = (gsem0, gsem1)
        isem = (isem0, isem1)
        has_extra = wid < n_extra
        chunk0 = jnp.where(has_extra, wid * per_hi,
                           n_extra * per_hi + (wid - n_extra) * per_lo)

        @pl.when(cid == 0)
        def _():
            pltpu.sync_copy(vals_hbm.at[pl.ds(r0, rpt)], acc.at[pl.ds(r0, rpt)])

        @pl.when(cid != 0)
        def _():
            pltpu.sync_copy(zeros_hbm.at[pl.ds(r0, rpt)], acc.at[pl.ds(r0, rpt)])

        def b(j):
            return pl.multiple_of((chunk0 + j) * CHUNK, 8)

        def make(j):
            k = j & 1
            g = pltpu.make_async_copy(vals_hbm.at[sidx[k]], rows[k], gsem[k])
            d = pltpu.make_async_copy(dst_hbm.at[pl.ds(b(j), CHUNK)],
                                      didx[k], isem[k])

            def go():
                pltpu.sync_copy(src_hbm.at[pl.ds(b(j), CHUNK)], sidx[k])
                g.start()
                d.start()

            return g, d, go

        def fire(j):
            g, d, go = make(j)
            if j < per_lo:
                go()
            else:
                pl.when(has_extra)(go)
            return g, d

        descs = {0: fire(0)}
        plsc.subcore_barrier()
        for j in range(per_hi):
            cur = j & 1
            if j + 1 < per_hi:
                descs[j + 1] = fire(j + 1)
            g, d_ = descs.pop(j)

            def drain(g=g, d_=d_, cur=cur):
                g.wait()
                d_.wait()
                pltpu.sync_copy(rows[cur], acc.at[didx[cur]], add=True)

            if j < per_lo:
                drain()
            else:
                pl.when(has_extra)(drain)

        plsc.subcore_barrier()
        pltpu.sync_copy(acc.at[pl.ds(r0, rpt)],
                        out_hbm.at[cid, pl.ds(r0, rpt)])

    return agg_kernel


def _make_agg_split(n_pad, w, e, chunk):
    CHUNK = chunk
    half = w // 2
    n_chunks = e // CHUNK
    per_lo = n_chunks // N_SUBCORES
    n_extra = n_chunks - per_lo * N_SUBCORES
    per_hi = per_lo + (1 if n_extra else 0)

    @functools.partial(
        pl.kernel,
        out_type=jax.ShapeDtypeStruct((N_CORES, n_pad, half), jnp.float32),
        mesh=_mesh(),
        scratch_types=[
            pltpu.VMEM((CHUNK,), jnp.int32),
            pltpu.VMEM((CHUNK,), jnp.int32),
            pltpu.VMEM((CHUNK,), jnp.int32),
            pltpu.VMEM((CHUNK,), jnp.int32),
            pltpu.VMEM((CHUNK, half), jnp.float32),
            pltpu.VMEM((CHUNK, half), jnp.float32),
            pltpu.VMEM_SHARED((n_pad, half), jnp.float32),
            pltpu.SemaphoreType.DMA,
            pltpu.SemaphoreType.DMA,
            pltpu.SemaphoreType.DMA,
            pltpu.SemaphoreType.DMA,
        ],
        compiler_params=_SC_PARAMS,
    )
    def agg_kernel(srcev_hbm, srcod_hbm, dst_hbm, valsv_hbm, zeros_hbm,
                   out_hbm, sidx0, sidx1, didx0, didx1, rows0, rows1, acc,
                   gsem0, gsem1, isem0, isem1):
        cid = lax.axis_index("c")
        sid = lax.axis_index("s")
        rpt = n_pad // N_SUBCORES
        r0 = sid * rpt
        sidx = (sidx0, sidx1)
        didx = (didx0, didx1)
        rows = (rows0, rows1)
        gsem = (gsem0, gsem1)
        isem = (isem0, isem1)
        has_extra = sid < n_extra
        chunk0 = jnp.where(has_extra, sid * per_hi,
                           n_extra * per_hi + (sid - n_extra) * per_lo)

        pltpu.sync_copy(zeros_hbm.at[pl.ds(r0, rpt)], acc.at[pl.ds(r0, rpt)])

        def b(j):
            return pl.multiple_of((chunk0 + j) * CHUNK, 8)

        def run(src_hbm):
            def make(j):
                k = j & 1
                g = pltpu.make_async_copy(valsv_hbm.at[sidx[k]], rows[k],
                                          gsem[k])
                d = pltpu.make_async_copy(dst_hbm.at[pl.ds(b(j), CHUNK)],
                                          didx[k], isem[k])

                def go():
                    pltpu.sync_copy(src_hbm.at[pl.ds(b(j), CHUNK)], sidx[k])
                    g.start()
                    d.start()

                return g, d, go

            def fire(j):
                g, d, go = make(j)
                if j < per_lo:
                    go()
                else:
                    pl.when(has_extra)(go)
                return g, d

            descs = {0: fire(0)}
            for j in range(per_hi):
                cur = j & 1
                if j + 1 < per_hi:
                    descs[j + 1] = fire(j + 1)
                g, d_ = descs.pop(j)

                def drain(g=g, d_=d_, cur=cur):
                    g.wait()
                    d_.wait()
                    pltpu.sync_copy(rows[cur], acc.at[didx[cur]], add=True)

                if j < per_lo:
                    drain()
                else:
                    pl.when(has_extra)(drain)

        plsc.subcore_barrier()
        pl.when(cid == 0)(lambda: run(srcev_hbm))
        pl.when(cid != 0)(lambda: run(srcod_hbm))
        plsc.subcore_barrier()
        pltpu.sync_copy(acc.at[pl.ds(r0, rpt)],
                        out_hbm.at[cid, pl.ds(r0, rpt)])

    return agg_kernel


def _dense1_body(x_ref, w_ref, deg_ref, g_ref, dinv_ref):
    dinv = lax.rsqrt(deg_ref[0, :, 0:1] + deg_ref[1, :, 0:1] + 1.0)
    h = jnp.dot(x_ref[...], w_ref[...], preferred_element_type=jnp.float32)
    g_ref[...] = h * dinv
    dinv_ref[...] = dinv


def _dense2_body(acc_ref, g1_ref, dinv_ref, w_ref, b_ref, g_ref):
    dinv = dinv_ref[...]
    s = jnp.concatenate([acc_ref[0], acc_ref[1]], axis=1) + g1_ref[...]
    h1 = jnp.maximum(s * dinv + b_ref[...], 0.0)
    h2 = jnp.dot(h1, w_ref[...], preferred_element_type=jnp.float32)
    g_ref[...] = h2 * dinv


def _dense3_body(acc_ref, dinv_ref, b_ref, out_ref):
    d_out = out_ref.shape[1]
    out_ref[...] = ((acc_ref[0, :, :d_out] + acc_ref[1, :, :d_out])
                    * dinv_ref[...] + b_ref[...])


def _dense1(n_pad, d_in, hidden, blk):
    grid = (n_pad // blk,)
    return pl.pallas_call(
        _dense1_body,
        grid=grid,
        in_specs=[
            pl.BlockSpec((blk, d_in), lambda i: (i, 0)),
            pl.BlockSpec((d_in, hidden), lambda i: (0, 0)),
            pl.BlockSpec((N_CORES, blk, 8), lambda i: (0, i, 0)),
        ],
        out_specs=[
            pl.BlockSpec((blk, hidden), lambda i: (i, 0)),
            pl.BlockSpec((blk, 1), lambda i: (i, 0)),
        ],
        out_shape=[
            jax.ShapeDtypeStruct((n_pad, hidden), jnp.float32),
            jax.ShapeDtypeStruct((n_pad, 1), jnp.float32),
        ],
    )


def _dense2(n_pad, hidden, d_out, blk):
    grid = (n_pad // blk,)
    return pl.pallas_call(
        _dense2_body,
        grid=grid,
        in_specs=[
            pl.BlockSpec((N_CORES, blk, hidden // 2), lambda i: (0, i, 0)),
            pl.BlockSpec((blk, hidden), lambda i: (i, 0)),
            pl.BlockSpec((blk, 1), lambda i: (i, 0)),
            pl.BlockSpec((hidden, d_out), lambda i: (0, 0)),
            pl.BlockSpec((1, hidden), lambda i: (0, 0)),
        ],
        out_specs=pl.BlockSpec((blk, d_out), lambda i: (i, 0)),
        out_shape=jax.ShapeDtypeStruct((n_pad, d_out), jnp.float32),
    )


def _dense3(n, n_pad, wp, d_out, blk):
    grid = (n_pad // blk,)
    return pl.pallas_call(
        _dense3_body,
        grid=grid,
        in_specs=[
            pl.BlockSpec((N_CORES, blk, wp), lambda i: (0, i, 0)),
            pl.BlockSpec((blk, 1), lambda i: (i, 0)),
            pl.BlockSpec((1, d_out), lambda i: (0, 0)),
        ],
        out_specs=pl.BlockSpec((blk, d_out), lambda i: (i, 0)),
        out_shape=jax.ShapeDtypeStruct((n, d_out), jnp.float32),
    )


def kernel(x, edge_index, W1, b1, W2, b2):
    n, d_in = x.shape
    hidden = W1.shape[1]
    d_out = W2.shape[1]
    e = edge_index.shape[1]

    blk = 2048
    n_pad = ((n + blk - 1) // blk) * blk
    wp = ((d_out + 7) // 8) * 8

    ei = edge_index.astype(jnp.int32)
    src1 = ei[0]
    dst1 = ei[1]

    x_p = jnp.pad(x, ((0, n_pad - n), (0, 0)))
    def pick_chunk(limit):
        for c in range(limit, 0, -8):
            if e % c == 0:
                return c
        return CHUNK

    ch_split = pick_chunk(1280)
    ch_thin = 2000 if e % (2000 * N_WORKERS) == 0 else pick_chunk(2000)

    W2_p = jnp.pad(W2, ((0, 0), (0, wp - d_out)))
    ones8 = jnp.ones((ch_thin, 8), jnp.float32)
    zdeg = jnp.zeros((n_pad, 8), jnp.float32)
    zeros_half = jnp.zeros((n_pad, hidden // 2), jnp.float32)
    zeros_o = jnp.zeros((n_pad, wp), jnp.float32)
    src_ev = src1 * 2
    src_od = src1 * 2 + 1

    deg = _make_deg(n_pad, e, ch_thin)(dst1, ones8, zdeg)
    g1, dinv = _dense1(n_pad, d_in, hidden, blk)(x_p, W1, deg)
    g1v = g1.reshape(2 * n_pad, hidden // 2)
    acc1 = _make_agg_split(n_pad, hidden, e, ch_split)(
        src_ev, src_od, dst1, g1v, zeros_half)
    g2 = _dense2(n_pad, hidden, wp, blk)(acc1, g1, dinv, W2_p,
                                         b1.reshape(1, hidden))
    acc2 = _make_agg(n_pad, wp, e, ch_thin)(src1, dst1, g2, zeros_o)
    return _dense3(n, n_pad, wp, d_out, blk)(acc2, dinv,
                                             b2.reshape(1, d_out))


# speedup vs baseline: 1.0012x; 1.0012x over previous
import functools

import jax
import jax.numpy as jnp
from jax import lax
from jax.experimental import pallas as pl
from jax.experimental.pallas import tpu as pltpu
from jax.experimental.pallas import tpu_sc as plsc

CHUNK = 400
N_CORES = 2
N_SUBCORES = 16
N_WORKERS = N_CORES * N_SUBCORES


def _mesh():
    return plsc.VectorSubcoreMesh(core_axis_name="c", subcore_axis_name="s",
                                  num_cores=N_CORES, num_subcores=N_SUBCORES)


_SC_PARAMS = pltpu.CompilerParams(use_tc_tiling_on_sc=False)


def _make_deg(n_pad, e, chunk):
    CHUNK = chunk
    per_w = e // CHUNK // N_WORKERS

    @functools.partial(
        pl.kernel,
        out_type=jax.ShapeDtypeStruct((N_CORES, n_pad, 8), jnp.float32),
        mesh=_mesh(),
        scratch_types=[
            pltpu.VMEM((CHUNK,), jnp.int32),
            pltpu.VMEM((CHUNK,), jnp.int32),
            pltpu.VMEM((CHUNK, 8), jnp.float32),
            pltpu.VMEM_SHARED((n_pad, 8), jnp.float32),
            pltpu.SemaphoreType.DMA,
            pltpu.SemaphoreType.DMA,
        ],
        compiler_params=_SC_PARAMS,
    )
    def deg_kernel(dst_hbm, ones_hbm, zcol_hbm, out_hbm,
                   didx0, didx1, ones_v, acc, isem0, isem1):
        cid = lax.axis_index("c")
        sid = lax.axis_index("s")
        wid = sid * N_CORES + cid
        rpt = n_pad // N_SUBCORES
        r0 = sid * rpt
        didx = (didx0, didx1)
        isem = (isem0, isem1)

        pltpu.sync_copy(zcol_hbm.at[pl.ds(r0, rpt)], acc.at[pl.ds(r0, rpt)])
        pltpu.sync_copy(ones_hbm, ones_v)

        def b(j):
            return pl.multiple_of((wid * per_w + j) * CHUNK, 8)

        descs = {0: pltpu.async_copy(dst_hbm.at[pl.ds(b(0), CHUNK)],
                                     didx[0], isem[0])}
        plsc.subcore_barrier()
        for j in range(per_w):
            cur = j & 1
            if j + 1 < per_w:
                descs[j + 1] = pltpu.async_copy(
                    dst_hbm.at[pl.ds(b(j + 1), CHUNK)], didx[1 - cur],
                    isem[1 - cur])
            descs.pop(j).wait()
            pltpu.sync_copy(ones_v, acc.at[didx[cur]], add=True)

        plsc.subcore_barrier()
        pltpu.sync_copy(acc.at[pl.ds(r0, rpt)],
                        out_hbm.at[cid, pl.ds(r0, rpt)])

    return deg_kernel


def _make_agg(n_pad, w, e, chunk):
    CHUNK = chunk
    n_chunks = e // CHUNK
    per_lo = n_chunks // N_WORKERS
    n_extra = n_chunks - per_lo * N_WORKERS
    per_hi = per_lo + (1 if n_extra else 0)

    @functools.partial(
        pl.kernel,
        out_type=jax.ShapeDtypeStruct((N_CORES, n_pad, w), jnp.float32),
        mesh=_mesh(),
        scratch_types=[
            pltpu.VMEM((CHUNK,), jnp.int32),
            pltpu.VMEM((CHUNK,), jnp.int32),
            pltpu.VMEM((CHUNK,), jnp.int32),
            pltpu.VMEM((CHUNK,), jnp.int32),
            pltpu.VMEM((CHUNK, w), jnp.float32),
            pltpu.VMEM((CHUNK, w), jnp.float32),
            pltpu.VMEM_SHARED((n_pad, w), jnp.float32),
            pltpu.SemaphoreType.DMA,
            pltpu.SemaphoreType.DMA,
            pltpu.SemaphoreType.DMA,
            pltpu.SemaphoreType.DMA,
        ],
        compiler_params=_SC_PARAMS,
    )
    def agg_kernel(src_hbm, dst_hbm, vals_hbm, zeros_hbm, out_hbm,
                   sidx0, sidx1, didx0, didx1, rows0, rows1, acc,
                   gsem0, gsem1, isem0, isem1):
        cid = lax.axis_index("c")
        sid = lax.axis_index("s")
        wid = sid * N_CORES + cid
        rpt = n_pad // N_SUBCORES
        r0 = sid * rpt
        sidx = (sidx0, sidx1)
        didx = (didx0, didx1)
        rows = (rows0, rows1)
        gsem = (gsem0, gsem1)
        isem = (isem0, isem1)
        has_extra = wid < n_extra
        chunk0 = jnp.where(has_extra, wid * per_hi,
                           n_extra * per_hi + (wid - n_extra) * per_lo)

        @pl.when(cid == 0)
        def _():
            pltpu.sync_copy(vals_hbm.at[pl.ds(r0, rpt)], acc.at[pl.ds(r0, rpt)])

        @pl.when(cid != 0)
        def _():
            pltpu.sync_copy(zeros_hbm.at[pl.ds(r0, rpt)], acc.at[pl.ds(r0, rpt)])

        def b(j):
            return pl.multiple_of((chunk0 + j) * CHUNK, 8)

        def make(j):
            k = j & 1
            g = pltpu.make_async_copy(vals_hbm.at[sidx[k]], rows[k], gsem[k])
            d = pltpu.make_async_copy(dst_hbm.at[pl.ds(b(j), CHUNK)],
                                      didx[k], isem[k])

            def go():
                pltpu.sync_copy(src_hbm.at[pl.ds(b(j), CHUNK)], sidx[k])
                g.start()
                d.start()

            return g, d, go

        def fire(j):
            g, d, go = make(j)
            if j < per_lo:
                go()
            else:
                pl.when(has_extra)(go)
            return g, d

        descs = {0: fire(0)}
        plsc.subcore_barrier()
        for j in range(per_hi):
            cur = j & 1
            if j + 1 < per_hi:
                descs[j + 1] = fire(j + 1)
            g, d_ = descs.pop(j)

            def drain(g=g, d_=d_, cur=cur):
                g.wait()
                d_.wait()
                pltpu.sync_copy(rows[cur], acc.at[didx[cur]], add=True)

            if j < per_lo:
                drain()
            else:
                pl.when(has_extra)(drain)

        plsc.subcore_barrier()
        pltpu.sync_copy(acc.at[pl.ds(r0, rpt)],
                        out_hbm.at[cid, pl.ds(r0, rpt)])

    return agg_kernel


def _make_agg_split(n_pad, w, e, chunk):
    CHUNK = chunk
    half = w // 2
    n_chunks = e // CHUNK
    per_lo = n_chunks // N_SUBCORES
    n_extra = n_chunks - per_lo * N_SUBCORES
    per_hi = per_lo + (1 if n_extra else 0)

    @functools.partial(
        pl.kernel,
        out_type=jax.ShapeDtypeStruct((N_CORES, n_pad, half), jnp.float32),
        mesh=_mesh(),
        scratch_types=[
            pltpu.VMEM((CHUNK,), jnp.int32),
            pltpu.VMEM((CHUNK,), jnp.int32),
            pltpu.VMEM((CHUNK,), jnp.int32),
            pltpu.VMEM((CHUNK,), jnp.int32),
            pltpu.VMEM((CHUNK, half), jnp.float32),
            pltpu.VMEM((CHUNK, half), jnp.float32),
            pltpu.VMEM_SHARED((n_pad, half), jnp.float32),
            pltpu.SemaphoreType.DMA,
            pltpu.SemaphoreType.DMA,
            pltpu.SemaphoreType.DMA,
            pltpu.SemaphoreType.DMA,
        ],
        compiler_params=_SC_PARAMS,
    )
    def agg_kernel(srcev_hbm, srcod_hbm, dst_hbm, valsv_hbm, zeros_hbm,
                   out_hbm, sidx0, sidx1, didx0, didx1, rows0, rows1, acc,
                   gsem0, gsem1, isem0, isem1):
        cid = lax.axis_index("c")
        sid = lax.axis_index("s")
        rpt = n_pad // N_SUBCORES
        r0 = sid * rpt
        sidx = (sidx0, sidx1)
        didx = (didx0, didx1)
        rows = (rows0, rows1)
        gsem = (gsem0, gsem1)
        isem = (isem0, isem1)
        has_extra = sid < n_extra
        chunk0 = jnp.where(has_extra, sid * per_hi,
                           n_extra * per_hi + (sid - n_extra) * per_lo)

        pltpu.sync_copy(zeros_hbm.at[pl.ds(r0, rpt)], acc.at[pl.ds(r0, rpt)])

        def b(j):
            return pl.multiple_of((chunk0 + j) * CHUNK, 8)

        def run(src_hbm):
            def make(j):
                k = j & 1
                g = pltpu.make_async_copy(valsv_hbm.at[sidx[k]], rows[k],
                                          gsem[k])
                d = pltpu.make_async_copy(dst_hbm.at[pl.ds(b(j), CHUNK)],
                                          didx[k], isem[k])

                def go():
                    pltpu.sync_copy(src_hbm.at[pl.ds(b(j), CHUNK)], sidx[k])
                    g.start()
                    d.start()

                return g, d, go

            def fire(j):
                g, d, go = make(j)
                if j < per_lo:
                    go()
                else:
                    pl.when(has_extra)(go)
                return g, d

            descs = {0: fire(0)}
            for j in range(per_hi):
                cur = j & 1
                if j + 1 < per_hi:
                    descs[j + 1] = fire(j + 1)
                g, d_ = descs.pop(j)

                def drain(g=g, d_=d_, cur=cur):
                    g.wait()
                    d_.wait()
                    pltpu.sync_copy(rows[cur], acc.at[didx[cur]], add=True)

                if j < per_lo:
                    drain()
                else:
                    pl.when(has_extra)(drain)

        plsc.subcore_barrier()
        pl.when(cid == 0)(lambda: run(srcev_hbm))
        pl.when(cid != 0)(lambda: run(srcod_hbm))
        plsc.subcore_barrier()
        pltpu.sync_copy(acc.at[pl.ds(r0, rpt)],
                        out_hbm.at[cid, pl.ds(r0, rpt)])

    return agg_kernel


def _dense1_body(x_ref, w_ref, deg_ref, g_ref, dinv_ref):
    dinv = lax.rsqrt(deg_ref[0, :, 0:1] + deg_ref[1, :, 0:1] + 1.0)
    h = jnp.dot(x_ref[...], w_ref[...], preferred_element_type=jnp.float32)
    g_ref[...] = h * dinv
    dinv_ref[...] = dinv


def _dense2_body(acc_ref, g1_ref, dinv_ref, w_ref, b_ref, g_ref):
    dinv = dinv_ref[...]
    s = jnp.concatenate([acc_ref[0], acc_ref[1]], axis=1) + g1_ref[...]
    h1 = jnp.maximum(s * dinv + b_ref[...], 0.0)
    h2 = jnp.dot(h1, w_ref[...], preferred_element_type=jnp.float32)
    g_ref[...] = h2 * dinv


def _dense3_body(acc_ref, dinv_ref, b_ref, out_ref):
    d_out = out_ref.shape[1]
    out_ref[...] = ((acc_ref[0, :, :d_out] + acc_ref[1, :, :d_out])
                    * dinv_ref[...] + b_ref[...])


def _dense1(n_pad, d_in, hidden, blk):
    grid = (n_pad // blk,)
    return pl.pallas_call(
        _dense1_body,
        grid=grid,
        in_specs=[
            pl.BlockSpec((blk, d_in), lambda i: (i, 0)),
            pl.BlockSpec((d_in, hidden), lambda i: (0, 0)),
            pl.BlockSpec((N_CORES, blk, 8), lambda i: (0, i, 0)),
        ],
        out_specs=[
            pl.BlockSpec((blk, hidden), lambda i: (i, 0)),
            pl.BlockSpec((blk, 1), lambda i: (i, 0)),
        ],
        out_shape=[
            jax.ShapeDtypeStruct((n_pad, hidden), jnp.float32),
            jax.ShapeDtypeStruct((n_pad, 1), jnp.float32),
        ],
    )


def _dense2(n_pad, hidden, d_out, blk):
    grid = (n_pad // blk,)
    return pl.pallas_call(
        _dense2_body,
        grid=grid,
        in_specs=[
            pl.BlockSpec((N_CORES, blk, hidden // 2), lambda i: (0, i, 0)),
            pl.BlockSpec((blk, hidden), lambda i: (i, 0)),
            pl.BlockSpec((blk, 1), lambda i: (i, 0)),
            pl.BlockSpec((hidden, d_out), lambda i: (0, 0)),
            pl.BlockSpec((1, hidden), lambda i: (0, 0)),
        ],
        out_specs=pl.BlockSpec((blk, d_out), lambda i: (i, 0)),
        out_shape=jax.ShapeDtypeStruct((n_pad, d_out), jnp.float32),
    )


def _dense3(n, n_pad, wp, d_out, blk):
    grid = (n_pad // blk,)
    return pl.pallas_call(
        _dense3_body,
        grid=grid,
        in_specs=[
            pl.BlockSpec((N_CORES, blk, wp), lambda i: (0, i, 0)),
            pl.BlockSpec((blk, 1), lambda i: (i, 0)),
            pl.BlockSpec((1, d_out), lambda i: (0, 0)),
        ],
        out_specs=pl.BlockSpec((blk, d_out), lambda i: (i, 0)),
        out_shape=jax.ShapeDtypeStruct((n, d_out), jnp.float32),
    )


def kernel(x, edge_index, W1, b1, W2, b2):
    n, d_in = x.shape
    hidden = W1.shape[1]
    d_out = W2.shape[1]
    e = edge_index.shape[1]

    blk = 2048
    n_pad = ((n + blk - 1) // blk) * blk
    wp = ((d_out + 7) // 8) * 8

    ei = edge_index.astype(jnp.int32)
    dst1 = jax.lax.optimization_barrier(ei[1])
    src1 = ei[0]

    x_p = jnp.pad(x, ((0, n_pad - n), (0, 0)))
    def pick_chunk(limit):
        for c in range(limit, 0, -8):
            if e % c == 0:
                return c
        return CHUNK

    ch_split = pick_chunk(1280)
    ch_thin = 2000 if e % (2000 * N_WORKERS) == 0 else pick_chunk(2000)

    W2_p = jnp.pad(W2, ((0, 0), (0, wp - d_out)))
    ones8 = jnp.ones((ch_thin, 8), jnp.float32)
    zdeg = jnp.zeros((n_pad, 8), jnp.float32)
    zeros_half = jnp.zeros((n_pad, hidden // 2), jnp.float32)
    zeros_o = jnp.zeros((n_pad, wp), jnp.float32)
    src_ev = src1 * 2
    src_od = src1 * 2 + 1

    deg = _make_deg(n_pad, e, ch_thin)(dst1, ones8, zdeg)
    g1, dinv = _dense1(n_pad, d_in, hidden, blk)(x_p, W1, deg)
    g1v = g1.reshape(2 * n_pad, hidden // 2)
    acc1 = _make_agg_split(n_pad, hidden, e, ch_split)(
        src_ev, src_od, dst1, g1v, zeros_half)
    g2 = _dense2(n_pad, hidden, wp, blk)(acc1, g1, dinv, W2_p,
                                         b1.reshape(1, hidden))
    acc2 = _make_agg(n_pad, wp, e, ch_thin)(src1, dst1, g2, zeros_o)
    return _dense3(n, n_pad, wp, d_out, blk)(acc2, dinv,
                                             b2.reshape(1, d_out))


# speedup vs baseline: 1.0087x; 1.0075x over previous
import functools

import jax
import jax.numpy as jnp
from jax import lax
from jax.experimental import pallas as pl
from jax.experimental.pallas import tpu as pltpu
from jax.experimental.pallas import tpu_sc as plsc

CHUNK = 400
N_CORES = 2
N_SUBCORES = 16
N_WORKERS = N_CORES * N_SUBCORES


def _mesh():
    return plsc.VectorSubcoreMesh(core_axis_name="c", subcore_axis_name="s",
                                  num_cores=N_CORES, num_subcores=N_SUBCORES)


_SC_PARAMS = pltpu.CompilerParams(use_tc_tiling_on_sc=False)


def _make_deg(n_pad, e, chunk):
    CHUNK = chunk
    per_w = e // CHUNK // N_WORKERS

    @functools.partial(
        pl.kernel,
        out_type=jax.ShapeDtypeStruct((N_CORES, n_pad, 8), jnp.float32),
        mesh=_mesh(),
        scratch_types=[
            pltpu.VMEM((CHUNK,), jnp.int32),
            pltpu.VMEM((CHUNK,), jnp.int32),
            pltpu.VMEM((CHUNK, 8), jnp.float32),
            pltpu.VMEM_SHARED((n_pad, 8), jnp.float32),
            pltpu.SemaphoreType.DMA,
            pltpu.SemaphoreType.DMA,
        ],
        compiler_params=_SC_PARAMS,
    )
    def deg_kernel(dst_hbm, ones_hbm, zcol_hbm, out_hbm,
                   didx0, didx1, ones_v, acc, isem0, isem1):
        cid = lax.axis_index("c")
        sid = lax.axis_index("s")
        wid = sid * N_CORES + cid
        rpt = n_pad // N_SUBCORES
        r0 = sid * rpt
        didx = (didx0, didx1)
        isem = (isem0, isem1)

        pltpu.sync_copy(zcol_hbm.at[pl.ds(r0, rpt)], acc.at[pl.ds(r0, rpt)])
        pltpu.sync_copy(ones_hbm, ones_v)

        def b(j):
            return pl.multiple_of((wid * per_w + j) * CHUNK, 8)

        descs = {0: pltpu.async_copy(dst_hbm.at[pl.ds(b(0), CHUNK)],
                                     didx[0], isem[0])}
        plsc.subcore_barrier()
        for j in range(per_w):
            cur = j & 1
            if j + 1 < per_w:
                descs[j + 1] = pltpu.async_copy(
                    dst_hbm.at[pl.ds(b(j + 1), CHUNK)], didx[1 - cur],
                    isem[1 - cur])
            descs.pop(j).wait()
            pltpu.sync_copy(ones_v, acc.at[didx[cur]], add=True)

        plsc.subcore_barrier()
        pltpu.sync_copy(acc.at[pl.ds(r0, rpt)],
                        out_hbm.at[cid, pl.ds(r0, rpt)])

    return deg_kernel


def _make_agg(n_pad, w, e, chunk):
    CHUNK = chunk
    n_chunks = e // CHUNK
    per_lo = n_chunks // N_WORKERS
    n_extra = n_chunks - per_lo * N_WORKERS
    per_hi = per_lo + (1 if n_extra else 0)

    @functools.partial(
        pl.kernel,
        out_type=jax.ShapeDtypeStruct((N_CORES, n_pad, w), jnp.float32),
        mesh=_mesh(),
        scratch_types=[
            pltpu.VMEM((CHUNK,), jnp.int32),
            pltpu.VMEM((CHUNK,), jnp.int32),
            pltpu.VMEM((CHUNK,), jnp.int32),
            pltpu.VMEM((CHUNK,), jnp.int32),
            pltpu.VMEM((CHUNK, w), jnp.float32),
            pltpu.VMEM((CHUNK, w), jnp.float32),
            pltpu.VMEM_SHARED((n_pad, w), jnp.float32),
            pltpu.SemaphoreType.DMA,
            pltpu.SemaphoreType.DMA,
            pltpu.SemaphoreType.DMA,
            pltpu.SemaphoreType.DMA,
        ],
        compiler_params=_SC_PARAMS,
    )
    def agg_kernel(src_hbm, dst_hbm, vals_hbm, zeros_hbm, out_hbm,
                   sidx0, sidx1, didx0, didx1, rows0, rows1, acc,
                   gsem0, gsem1, isem0, isem1):
        cid = lax.axis_index("c")
        sid = lax.axis_index("s")
        wid = sid * N_CORES + cid
        rpt = n_pad // N_SUBCORES
        r0 = sid * rpt
        sidx = (sidx0, sidx1)
        didx = (didx0, didx1)
        rows = (rows0, rows1)
        gsem = (gsem0, gsem1)
        isem = (isem0, isem1)
        has_extra = wid < n_extra
        chunk0 = jnp.where(has_extra, wid * per_hi,
                           n_extra * per_hi + (wid - n_extra) * per_lo)

        @pl.when(cid == 0)
        def _():
            pltpu.sync_copy(vals_hbm.at[pl.ds(r0, rpt)], acc.at[pl.ds(r0, rpt)])

        @pl.when(cid != 0)
        def _():
            pltpu.sync_copy(zeros_hbm.at[pl.ds(r0, rpt)], acc.at[pl.ds(r0, rpt)])

        def b(j):
            return pl.multiple_of((chunk0 + j) * CHUNK, 8)

        def make(j):
            k = j & 1
            g = pltpu.make_async_copy(vals_hbm.at[sidx[k]], rows[k], gsem[k])
            d = pltpu.make_async_copy(dst_hbm.at[pl.ds(b(j), CHUNK)],
                                      didx[k], isem[k])

            def go():
                pltpu.sync_copy(src_hbm.at[pl.ds(b(j), CHUNK)], sidx[k])
                g.start()
                d.start()

            return g, d, go

        def fire(j):
            g, d, go = make(j)
            if j < per_lo:
                go()
            else:
                pl.when(has_extra)(go)
            return g, d

        descs = {0: fire(0)}
        plsc.subcore_barrier()
        for j in range(per_hi):
            cur = j & 1
            if j + 1 < per_hi:
                descs[j + 1] = fire(j + 1)
            g, d_ = descs.pop(j)

            def drain(g=g, d_=d_, cur=cur):
                g.wait()
                d_.wait()
                pltpu.sync_copy(rows[cur], acc.at[didx[cur]], add=True)

            if j < per_lo:
                drain()
            else:
                pl.when(has_extra)(drain)

        plsc.subcore_barrier()
        pltpu.sync_copy(acc.at[pl.ds(r0, rpt)],
                        out_hbm.at[cid, pl.ds(r0, rpt)])

    return agg_kernel


def _dense1_body(x_ref, w_ref, deg_ref, g_ref, dinv_ref):
    dinv = lax.rsqrt(deg_ref[0, :, 0:1] + deg_ref[1, :, 0:1] + 1.0)
    h = jnp.dot(x_ref[...], w_ref[...], preferred_element_type=jnp.float32)
    g_ref[...] = h * dinv
    dinv_ref[...] = dinv


def _dense2_body(acc_ref, dinv_ref, w_ref, b_ref, g_ref):
    dinv = dinv_ref[...]
    s = acc_ref[0] + acc_ref[1]
    h1 = jnp.maximum(s * dinv + b_ref[...], 0.0)
    h2 = jnp.dot(h1, w_ref[...], preferred_element_type=jnp.float32)
    g_ref[...] = h2 * dinv


def _dense3_body(acc_ref, dinv_ref, b_ref, out_ref):
    d_out = out_ref.shape[1]
    out_ref[...] = ((acc_ref[0, :, :d_out] + acc_ref[1, :, :d_out])
                    * dinv_ref[...] + b_ref[...])


def _dense1(n_pad, d_in, hidden, blk):
    grid = (n_pad // blk,)
    return pl.pallas_call(
        _dense1_body,
        grid=grid,
        in_specs=[
            pl.BlockSpec((blk, d_in), lambda i: (i, 0)),
            pl.BlockSpec((d_in, hidden), lambda i: (0, 0)),
            pl.BlockSpec((N_CORES, blk, 8), lambda i: (0, i, 0)),
        ],
        out_specs=[
            pl.BlockSpec((blk, hidden), lambda i: (i, 0)),
            pl.BlockSpec((blk, 1), lambda i: (i, 0)),
        ],
        out_shape=[
            jax.ShapeDtypeStruct((n_pad, hidden), jnp.float32),
            jax.ShapeDtypeStruct((n_pad, 1), jnp.float32),
        ],
    )


def _dense2(n_pad, hidden, d_out, blk):
    grid = (n_pad // blk,)
    return pl.pallas_call(
        _dense2_body,
        grid=grid,
        in_specs=[
            pl.BlockSpec((N_CORES, blk, hidden), lambda i: (0, i, 0)),
            pl.BlockSpec((blk, 1), lambda i: (i, 0)),
            pl.BlockSpec((hidden, d_out), lambda i: (0, 0)),
            pl.BlockSpec((1, hidden), lambda i: (0, 0)),
        ],
        out_specs=pl.BlockSpec((blk, d_out), lambda i: (i, 0)),
        out_shape=jax.ShapeDtypeStruct((n_pad, d_out), jnp.float32),
    )


def _dense3(n, n_pad, wp, d_out, blk):
    grid = (n_pad // blk,)
    return pl.pallas_call(
        _dense3_body,
        grid=grid,
        in_specs=[
            pl.BlockSpec((N_CORES, blk, wp), lambda i: (0, i, 0)),
            pl.BlockSpec((blk, 1), lambda i: (i, 0)),
            pl.BlockSpec((1, d_out), lambda i: (0, 0)),
        ],
        out_specs=pl.BlockSpec((blk, d_out), lambda i: (i, 0)),
        out_shape=jax.ShapeDtypeStruct((n, d_out), jnp.float32),
    )


def kernel(x, edge_index, W1, b1, W2, b2):
    n, d_in = x.shape
    hidden = W1.shape[1]
    d_out = W2.shape[1]
    e = edge_index.shape[1]

    blk = 2048
    n_pad = ((n + blk - 1) // blk) * blk
    wp = ((d_out + 7) // 8) * 8

    ei = edge_index.astype(jnp.int32)
    dst1 = jax.lax.optimization_barrier(ei[1])
    src1 = ei[0]

    x_p = jnp.pad(x, ((0, n_pad - n), (0, 0)))
    def pick_chunk(limit):
        for c in range(limit, 0, -8):
            if e % c == 0:
                return c
        return CHUNK

    ch_wide = pick_chunk(640)
    ch_thin = 2000 if e % (2000 * N_WORKERS) == 0 else pick_chunk(2000)

    W2_p = jnp.pad(W2, ((0, 0), (0, wp - d_out)))
    ones8 = jnp.ones((ch_thin, 8), jnp.float32)
    zdeg = jnp.zeros((n_pad, 8), jnp.float32)
    zeros_h = jnp.zeros((n_pad, hidden), jnp.float32)
    zeros_o = jnp.zeros((n_pad, wp), jnp.float32)

    deg = _make_deg(n_pad, e, ch_thin)(dst1, ones8, zdeg)
    g1, dinv = _dense1(n_pad, d_in, hidden, blk)(x_p, W1, deg)
    acc1 = _make_agg(n_pad, hidden, e, ch_wide)(src1, dst1, g1, zeros_h)
    g2 = _dense2(n_pad, hidden, wp, blk)(acc1, dinv, W2_p,
                                         b1.reshape(1, hidden))
    acc2 = _make_agg(n_pad, wp, e, ch_thin)(src1, dst1, g2, zeros_o)
    return _dense3(n, n_pad, wp, d_out, blk)(acc2, dinv,
                                             b2.reshape(1, d_out))
